# baseline (speedup 1.0000x reference)
import functools

import jax
import jax.numpy as jnp
from jax import lax
from jax.experimental import pallas as pl
from jax.experimental.pallas import tpu as pltpu
from jax.experimental.pallas import tpu_sc as plsc

N = 10000
E = 320000
D = 128
NS = 16
NC = 2
NW = NC * NS

NP = N + 8
CHUNK = 128
TILE_E = 10240
TILE_CHUNKS = TILE_E // CHUNK
HALF_CHUNKS = TILE_CHUNKS // 2

_mesh = plsc.VectorSubcoreMesh(
    core_axis_name="c", subcore_axis_name="s", num_cores=NC, num_subcores=NS
)

_f32 = jnp.float32


@functools.partial(
    pl.kernel,
    out_type=jax.ShapeDtypeStruct((NC, NP), _f32),
    mesh=_mesh,
    scratch_types=[
        pltpu.VMEM((TILE_CHUNKS, CHUNK), jnp.int32),
        pltpu.VMEM((CHUNK,), _f32),
        pltpu.VMEM_SHARED((NP,), _f32),
        pltpu.SemaphoreType.DMA,
    ],
)
def _deg_kernel(dst_hbm, ones_hbm, zero_hbm, out_hbm, idx_v, ones_v, deg_sh, sem):
    cid = lax.axis_index("c")
    sid = lax.axis_index("s")
    wid = cid * NS + sid

    @pl.when(sid == 0)
    def _():
        pltpu.sync_copy(zero_hbm, deg_sh)

    pltpu.sync_copy(ones_hbm, ones_v)
    pltpu.sync_copy(dst_hbm.at[wid], idx_v)
    plsc.subcore_barrier()

    def body(j, carry):
        pltpu.sync_copy(ones_v, deg_sh.at[idx_v.at[j]], add=True)
        return carry

    lax.fori_loop(0, TILE_CHUNKS, body, 0)
    plsc.subcore_barrier()

    @pl.when(sid == 0)
    def _():
        pltpu.sync_copy(deg_sh, out_hbm.at[cid])


@functools.partial(
    pl.kernel,
    out_type=jax.ShapeDtypeStruct((NC, NP, D), _f32),
    mesh=_mesh,
    scratch_types=[
        pltpu.VMEM((HALF_CHUNKS, CHUNK), jnp.int32),
        pltpu.VMEM((HALF_CHUNKS, CHUNK), jnp.int32),
        pltpu.VMEM((CHUNK, D), _f32),
        pltpu.VMEM((CHUNK, D), _f32),
        pltpu.VMEM_SHARED((NP, D), _f32),
        pltpu.SemaphoreType.DMA,
        pltpu.SemaphoreType.DMA,
    ],
)
def _edge_kernel(g_hbm, src_hbm, dst_hbm, zero_hbm, out_hbm,
                 src_v, dst_v, rows0_v, rows1_v, acc_sh, s0, s1):
    cid = lax.axis_index("c")
    sid = lax.axis_index("s")
    wid = cid * NS + sid
    sems = (s0, s1)
    rows = (rows0_v, rows1_v)

    @pl.when(sid == 0)
    def _():
        pltpu.sync_copy(zero_hbm, acc_sh)

    def _gather(j, b):
        pltpu.async_copy(g_hbm.at[src_v.at[j]], rows[b], sems[b])

    def _gwait(b):
        pltpu.make_async_copy(zero_hbm.at[pl.ds(0, CHUNK)], rows[b],
                              sems[b]).wait()

    def _scat(j, b):
        pltpu.sync_copy(rows[b], acc_sh.at[dst_v.at[j]], add=True)

    for p in range(2):
        pltpu.sync_copy(src_hbm.at[wid * 2 + p], src_v)
        pltpu.sync_copy(dst_hbm.at[wid * 2 + p], dst_v)
        if p == 0:
            plsc.subcore_barrier()

        def body(j, carry):
            pltpu.async_copy(g_hbm.at[src_v.at[j]], rows0_v, s0).wait()
            _scat(j, 0)
            return carry

        lax.fori_loop(0, HALF_CHUNKS, body, 0)

    plsc.subcore_barrier()

    @pl.when(sid == 0)
    def _():
        pltpu.sync_copy(acc_sh, out_hbm.at[cid])


def _tc1_body(x_ref, we_ref, be_ref, w0_ref, degp_ref, dinv_ref, g_ref):
    deg = degp_ref[0, :N] + degp_ref[1, :N] + 1.0
    dinv = lax.rsqrt(deg)
    dinv_ref[...] = dinv
    h0 = jnp.dot(x_ref[...], we_ref[...], preferred_element_type=_f32)
    h0 = h0 + be_ref[...]
    g_ref[...] = dinv * jnp.dot(h0, w0_ref[...], preferred_element_type=_f32)


def _tc_mid_body(has_bn, ap_ref, g_ref, dinv_ref, b_ref, w_ref, *rest):
    if has_bn:
        bng_ref, bnb_ref, o_ref = rest
    else:
        (o_ref,) = rest
    dinv = dinv_ref[...]
    a = ap_ref[0, :N] + ap_ref[1, :N]
    t = dinv * (a + g_ref[...]) + b_ref[...]
    if has_bn:
        t = t * bng_ref[...] + bnb_ref[...]
    t = jnp.maximum(t, 0.0)
    o_ref[...] = dinv * jnp.dot(t, w_ref[...], preferred_element_type=_f32)


def _tc_last_body(ap_ref, g_ref, dinv_ref, b_ref, bng_ref, bnb_ref, out_ref):
    dinv = dinv_ref[...]
    a = ap_ref[0, :N] + ap_ref[1, :N]
    t = dinv * (a + g_ref[...]) + b_ref[...]
    out_ref[...] = t * bng_ref[...] + bnb_ref[...]


_BN_SCALE = 1.0 / (1.0 + 1e-5) ** 0.5

_nd = jax.ShapeDtypeStruct((N, D), _f32)


def kernel(x, edge_index, batch, edge_attr, We, be, W00, b00, W01, b01,
           W10, b10, W11, b11, bn0_g, bn0_b, bn1_g, bn1_b):
    del batch, edge_attr
    src = edge_index[0]
    dst = edge_index[1]

    pad = ((0, 0), (0, TILE_E - E // NW))
    src_p = jnp.pad(src.reshape(NW, E // NW), pad, constant_values=0)
    dst_p = jnp.pad(dst.reshape(NW, E // NW), pad, constant_values=N)
    src_h = src_p.reshape(NW * 2, HALF_CHUNKS, CHUNK)
    dst_h = dst_p.reshape(NW * 2, HALF_CHUNKS, CHUNK)
    dst_f = dst_p.reshape(NW, TILE_CHUNKS, CHUNK)

    ones_c = jnp.ones((CHUNK,), _f32)
    zero_n = jnp.zeros((NP,), _f32)
    zero_nd = jnp.zeros((NP, D), _f32)

    be2 = be.reshape(1, D)
    b00_2 = b00.reshape(1, D)
    b01_2 = b01.reshape(1, D)
    b10_2 = b10.reshape(1, D)
    b11_2 = b11.reshape(1, D)
    bn0g2 = (bn0_g * _BN_SCALE).reshape(1, D)
    bn0b2 = bn0_b.reshape(1, D)
    bn1g2 = (bn1_g * _BN_SCALE).reshape(1, D)
    bn1b2 = bn1_b.reshape(1, D)

    deg_parts = _deg_kernel(dst_f, ones_c, zero_n)
    degp = deg_parts.reshape(NC, NP, 1)

    dinv, g1 = pl.pallas_call(
        _tc1_body,
        out_shape=(jax.ShapeDtypeStruct((N, 1), _f32), _nd),
    )(x, We, be2, W00, degp)

    a1 = _edge_kernel(g1, src_h, dst_h, zero_nd)

    g2 = pl.pallas_call(
        functools.partial(_tc_mid_body, False), out_shape=_nd,
    )(a1, g1, dinv, b00_2, W01)

    a2 = _edge_kernel(g2, src_h, dst_h, zero_nd)

    g3 = pl.pallas_call(
        functools.partial(_tc_mid_body, True), out_shape=_nd,
    )(a2, g2, dinv, b01_2, W10, bn0g2, bn0b2)

    a3 = _edge_kernel(g3, src_h, dst_h, zero_nd)

    g4 = pl.pallas_call(
        functools.partial(_tc_mid_body, False), out_shape=_nd,
    )(a3, g3, dinv, b10_2, W11)

    a4 = _edge_kernel(g4, src_h, dst_h, zero_nd)

    out = pl.pallas_call(
        _tc_last_body, out_shape=_nd,
    )(a4, g4, dinv, b11_2, bn1g2, bn1b2)
    return out


# speedup vs baseline: 3.0819x; 3.0819x over previous
import functools

import jax
import jax.numpy as jnp
from jax import lax
from jax.experimental import pallas as pl
from jax.experimental.pallas import tpu as pltpu
from jax.experimental.pallas import tpu_sc as plsc

N = 10000
E = 320000
D = 128
NS = 16
NC = 2
NW = NC * NS

NP = N
CHUNK = 125
TILE_E = E // NW
TILE_CHUNKS = TILE_E // CHUNK
HALF_CHUNKS = TILE_CHUNKS // 2

_mesh = plsc.VectorSubcoreMesh(
    core_axis_name="c", subcore_axis_name="s", num_cores=NC, num_subcores=NS
)

_f32 = jnp.float32


@functools.partial(
    pl.kernel,
    out_type=jax.ShapeDtypeStruct((NC, NP), _f32),
    mesh=_mesh,
    scratch_types=[
        pltpu.VMEM((TILE_CHUNKS, CHUNK), jnp.int32),
        pltpu.VMEM((CHUNK,), _f32),
        pltpu.VMEM_SHARED((NP,), _f32),
        pltpu.SemaphoreType.DMA,
    ],
)
def _deg_kernel(dst_hbm, ones_hbm, zero_hbm, out_hbm, idx_v, ones_v, deg_sh, sem):
    cid = lax.axis_index("c")
    sid = lax.axis_index("s")
    wid = cid * NS + sid

    @pl.when(sid == 0)
    def _():
        pltpu.sync_copy(zero_hbm, deg_sh)

    pltpu.sync_copy(ones_hbm, ones_v)
    pltpu.sync_copy(dst_hbm.at[wid], idx_v)
    plsc.subcore_barrier()

    def body(j, carry):
        pltpu.sync_copy(ones_v, deg_sh.at[idx_v.at[j]], add=True)
        return carry

    lax.fori_loop(0, TILE_CHUNKS, body, 0)
    plsc.subcore_barrier()

    @pl.when(sid == 0)
    def _():
        pltpu.sync_copy(deg_sh, out_hbm.at[cid])


@functools.partial(
    pl.kernel,
    out_type=jax.ShapeDtypeStruct((NC, NP, D), _f32),
    mesh=_mesh,
    scratch_types=[
        pltpu.VMEM((HALF_CHUNKS, CHUNK), jnp.int32),
        pltpu.VMEM((HALF_CHUNKS, CHUNK), jnp.int32),
        pltpu.VMEM((CHUNK, D), _f32),
        pltpu.VMEM((CHUNK, D), _f32),
        pltpu.VMEM_SHARED((NP, D), _f32),
        pltpu.SemaphoreType.DMA,
        pltpu.SemaphoreType.DMA,
    ],
)
def _edge_kernel(g_hbm, src_hbm, dst_hbm, zero_hbm, out_hbm,
                 src_v, dst_v, rows0_v, rows1_v, acc_sh, s0, s1):
    cid = lax.axis_index("c")
    sid = lax.axis_index("s")
    wid = cid * NS + sid
    sems = (s0, s1)
    rows = (rows0_v, rows1_v)

    @pl.when(sid == 0)
    def _():
        pltpu.sync_copy(zero_hbm, acc_sh)

    def _gather(j, b):
        pltpu.async_copy(g_hbm.at[src_v.at[j]], rows[b], sems[b])

    def _gwait(b):
        pltpu.make_async_copy(g_hbm.at[src_v.at[0]], rows[b],
                              sems[b]).wait()

    def _scat(j, b):
        pltpu.sync_copy(rows[b], acc_sh.at[dst_v.at[j]], add=True)

    for p in range(2):
        pltpu.sync_copy(src_hbm.at[wid * 2 + p], src_v)
        pltpu.sync_copy(dst_hbm.at[wid * 2 + p], dst_v)
        if p == 0:
            plsc.subcore_barrier()
        _gather(0, 0)

        def body(g, carry):
            j = 2 * g
            _gwait(0)
            _gather(j + 1, 1)
            _scat(j, 0)
            _gwait(1)

            @pl.when(g < HALF_CHUNKS // 2 - 1)
            def _():
                _gather(j + 2, 0)

            _scat(j + 1, 1)
            return carry

        lax.fori_loop(0, HALF_CHUNKS // 2, body, 0)

    plsc.subcore_barrier()

    @pl.when(sid == 0)
    def _():
        pltpu.sync_copy(acc_sh, out_hbm.at[cid])


def _tc1_body(x_ref, we_ref, be_ref, w0_ref, degp_ref, dinv_ref, g_ref):
    deg = degp_ref[0, :N] + degp_ref[1, :N] + 1.0
    dinv = lax.rsqrt(deg)
    dinv_ref[...] = dinv
    h0 = jnp.dot(x_ref[...], we_ref[...], preferred_element_type=_f32)
    h0 = h0 + be_ref[...]
    g_ref[...] = dinv * jnp.dot(h0, w0_ref[...], preferred_element_type=_f32)


def _tc_mid_body(has_bn, ap_ref, g_ref, dinv_ref, b_ref, w_ref, *rest):
    if has_bn:
        bng_ref, bnb_ref, o_ref = rest
    else:
        (o_ref,) = rest
    dinv = dinv_ref[...]
    a = ap_ref[0, :N] + ap_ref[1, :N]
    t = dinv * (a + g_ref[...]) + b_ref[...]
    if has_bn:
        t = t * bng_ref[...] + bnb_ref[...]
    t = jnp.maximum(t, 0.0)
    o_ref[...] = dinv * jnp.dot(t, w_ref[...], preferred_element_type=_f32)


def _tc_last_body(ap_ref, g_ref, dinv_ref, b_ref, bng_ref, bnb_ref, out_ref):
    dinv = dinv_ref[...]
    a = ap_ref[0, :N] + ap_ref[1, :N]
    t = dinv * (a + g_ref[...]) + b_ref[...]
    out_ref[...] = t * bng_ref[...] + bnb_ref[...]


_BN_SCALE = 1.0 / (1.0 + 1e-5) ** 0.5

_nd = jax.ShapeDtypeStruct((N, D), _f32)


def kernel(x, edge_index, batch, edge_attr, We, be, W00, b00, W01, b01,
           W10, b10, W11, b11, bn0_g, bn0_b, bn1_g, bn1_b):
    del batch, edge_attr
    src = edge_index[0]
    dst = edge_index[1]

    src_h = src.reshape(NW * 2, HALF_CHUNKS, CHUNK)
    dst_h = dst.reshape(NW * 2, HALF_CHUNKS, CHUNK)
    dst_f = dst.reshape(NW, TILE_CHUNKS, CHUNK)

    ones_c = jnp.ones((CHUNK,), _f32)
    zero_n = jnp.zeros((NP,), _f32)
    zero_nd = jnp.zeros((NP, D), _f32)

    be2 = be.reshape(1, D)
    b00_2 = b00.reshape(1, D)
    b01_2 = b01.reshape(1, D)
    b10_2 = b10.reshape(1, D)
    b11_2 = b11.reshape(1, D)
    bn0g2 = (bn0_g * _BN_SCALE).reshape(1, D)
    bn0b2 = bn0_b.reshape(1, D)
    bn1g2 = (bn1_g * _BN_SCALE).reshape(1, D)
    bn1b2 = bn1_b.reshape(1, D)

    deg_parts = _deg_kernel(dst_f, ones_c, zero_n)
    degp = deg_parts.reshape(NC, NP, 1)

    dinv, g1 = pl.pallas_call(
        _tc1_body,
        out_shape=(jax.ShapeDtypeStruct((N, 1), _f32), _nd),
    )(x, We, be2, W00, degp)

    a1 = _edge_kernel(g1, src_h, dst_h, zero_nd)

    g2 = pl.pallas_call(
        functools.partial(_tc_mid_body, False), out_shape=_nd,
    )(a1, g1, dinv, b00_2, W01)

    a2 = _edge_kernel(g2, src_h, dst_h, zero_nd)

    g3 = pl.pallas_call(
        functools.partial(_tc_mid_body, True), out_shape=_nd,
    )(a2, g2, dinv, b01_2, W10, bn0g2, bn0b2)

    a3 = _edge_kernel(g3, src_h, dst_h, zero_nd)

    g4 = pl.pallas_call(
        functools.partial(_tc_mid_body, False), out_shape=_nd,
    )(a3, g3, dinv, b10_2, W11)

    a4 = _edge_kernel(g4, src_h, dst_h, zero_nd)

    out = pl.pallas_call(
        _tc_last_body, out_shape=_nd,
    )(a4, g4, dinv, b11_2, bn1g2, bn1b2)
    return out


# speedup vs baseline: 3.0886x; 1.0022x over previous
import functools

import jax
import jax.numpy as jnp
from jax import lax
from jax.experimental import pallas as pl
from jax.experimental.pallas import tpu as pltpu
from jax.experimental.pallas import tpu_sc as plsc

N = 10000
E = 320000
D = 128
NS = 16
NC = 2
NW = NC * NS

NP = N
CHUNK = 125
TILE_E = E // NW
TILE_CHUNKS = TILE_E // CHUNK
HALF_CHUNKS = TILE_CHUNKS // 2

_mesh = plsc.VectorSubcoreMesh(
    core_axis_name="c", subcore_axis_name="s", num_cores=NC, num_subcores=NS
)

_f32 = jnp.float32


@functools.partial(
    pl.kernel,
    out_type=jax.ShapeDtypeStruct((NC, NP), _f32),
    mesh=_mesh,
    scratch_types=[
        pltpu.VMEM((TILE_CHUNKS, CHUNK), jnp.int32),
        pltpu.VMEM((CHUNK,), _f32),
        pltpu.VMEM_SHARED((NP,), _f32),
        pltpu.SemaphoreType.DMA,
    ],
)
def _deg_kernel(dst_hbm, ones_hbm, zero_hbm, out_hbm, idx_v, ones_v, deg_sh, sem):
    cid = lax.axis_index("c")
    sid = lax.axis_index("s")
    wid = cid * NS + sid

    @pl.when(sid == 0)
    def _():
        pltpu.sync_copy(zero_hbm, deg_sh)

    pltpu.sync_copy(ones_hbm, ones_v)
    pltpu.sync_copy(dst_hbm.at[wid], idx_v)
    plsc.subcore_barrier()

    def body(j, carry):
        pltpu.sync_copy(ones_v, deg_sh.at[idx_v.at[j]], add=True)
        return carry

    lax.fori_loop(0, TILE_CHUNKS, body, 0)
    plsc.subcore_barrier()

    @pl.when(sid == 0)
    def _():
        pltpu.sync_copy(deg_sh, out_hbm.at[cid])


@functools.partial(
    pl.kernel,
    out_type=jax.ShapeDtypeStruct((NC * NP, D), _f32),
    mesh=_mesh,
    scratch_types=[
        pltpu.VMEM((HALF_CHUNKS, CHUNK), jnp.int32),
        pltpu.VMEM((HALF_CHUNKS, CHUNK), jnp.int32),
        pltpu.VMEM((CHUNK, D), _f32),
        pltpu.VMEM((CHUNK, D), _f32),
        pltpu.VMEM_SHARED((NP, D), _f32),
        pltpu.SemaphoreType.DMA,
        pltpu.SemaphoreType.DMA,
    ],
)
def _edge_kernel(g_hbm, src_hbm, dst_hbm, zero_hbm, out_hbm,
                 src_v, dst_v, rows0_v, rows1_v, acc_sh, s0, s1):
    cid = lax.axis_index("c")
    sid = lax.axis_index("s")
    wid = cid * NS + sid
    sems = (s0, s1)
    rows = (rows0_v, rows1_v)

    @pl.when(sid < 10)
    def _():
        pltpu.sync_copy(zero_hbm.at[pl.ds(sid * 1000, 1000)],
                        acc_sh.at[pl.ds(sid * 1000, 1000)])

    def _gather(j, b):
        pltpu.async_copy(g_hbm.at[src_v.at[j]], rows[b], sems[b])

    def _gwait(b):
        pltpu.make_async_copy(g_hbm.at[src_v.at[0]], rows[b],
                              sems[b]).wait()

    def _scat(j, b):
        pltpu.sync_copy(rows[b], acc_sh.at[dst_v.at[j]], add=True)

    for p in range(2):
        pltpu.sync_copy(src_hbm.at[wid * 2 + p], src_v)
        pltpu.sync_copy(dst_hbm.at[wid * 2 + p], dst_v)
        if p == 0:
            plsc.subcore_barrier()
        _gather(0, 0)

        def body(g, carry):
            j = 2 * g
            _gwait(0)
            _gather(j + 1, 1)
            _scat(j, 0)
            _gwait(1)

            @pl.when(g < HALF_CHUNKS // 2 - 1)
            def _():
                _gather(j + 2, 0)

            _scat(j + 1, 1)
            return carry

        lax.fori_loop(0, HALF_CHUNKS // 2, body, 0)

    plsc.subcore_barrier()

    @pl.when(sid < 10)
    def _():
        pltpu.sync_copy(acc_sh.at[pl.ds(sid * 1000, 1000)],
                        out_hbm.at[pl.ds(cid * NP + sid * 1000, 1000)])


def _tc1_body(x_ref, we_ref, be_ref, w0_ref, degp_ref, dinv_ref, g_ref):
    deg = degp_ref[0, :N] + degp_ref[1, :N] + 1.0
    dinv = lax.rsqrt(deg)
    dinv_ref[...] = dinv
    h0 = jnp.dot(x_ref[...], we_ref[...], preferred_element_type=_f32)
    h0 = h0 + be_ref[...]
    g_ref[...] = dinv * jnp.dot(h0, w0_ref[...], preferred_element_type=_f32)


def _tc_mid_body(has_bn, ap_ref, g_ref, dinv_ref, b_ref, w_ref, *rest):
    if has_bn:
        bng_ref, bnb_ref, o_ref = rest
    else:
        (o_ref,) = rest
    dinv = dinv_ref[...]
    a = ap_ref[0, :N] + ap_ref[1, :N]
    t = dinv * (a + g_ref[...]) + b_ref[...]
    if has_bn:
        t = t * bng_ref[...] + bnb_ref[...]
    t = jnp.maximum(t, 0.0)
    o_ref[...] = dinv * jnp.dot(t, w_ref[...], preferred_element_type=_f32)


def _tc_last_body(ap_ref, g_ref, dinv_ref, b_ref, bng_ref, bnb_ref, out_ref):
    dinv = dinv_ref[...]
    a = ap_ref[0, :N] + ap_ref[1, :N]
    t = dinv * (a + g_ref[...]) + b_ref[...]
    out_ref[...] = t * bng_ref[...] + bnb_ref[...]


_BN_SCALE = 1.0 / (1.0 + 1e-5) ** 0.5

_nd = jax.ShapeDtypeStruct((N, D), _f32)


def kernel(x, edge_index, batch, edge_attr, We, be, W00, b00, W01, b01,
           W10, b10, W11, b11, bn0_g, bn0_b, bn1_g, bn1_b):
    del batch, edge_attr
    src = edge_index[0]
    dst = edge_index[1]

    src_h = src.reshape(NW * 2, HALF_CHUNKS, CHUNK)
    dst_h = dst.reshape(NW * 2, HALF_CHUNKS, CHUNK)
    dst_f = dst.reshape(NW, TILE_CHUNKS, CHUNK)

    ones_c = jnp.ones((CHUNK,), _f32)
    zero_n = jnp.zeros((NP,), _f32)
    zero_nd = jnp.zeros((NP, D), _f32)

    be2 = be.reshape(1, D)
    b00_2 = b00.reshape(1, D)
    b01_2 = b01.reshape(1, D)
    b10_2 = b10.reshape(1, D)
    b11_2 = b11.reshape(1, D)
    bn0g2 = (bn0_g * _BN_SCALE).reshape(1, D)
    bn0b2 = bn0_b.reshape(1, D)
    bn1g2 = (bn1_g * _BN_SCALE).reshape(1, D)
    bn1b2 = bn1_b.reshape(1, D)

    deg_parts = _deg_kernel(dst_f, ones_c, zero_n)
    degp = deg_parts.reshape(NC, NP, 1)

    dinv, g1 = pl.pallas_call(
        _tc1_body,
        out_shape=(jax.ShapeDtypeStruct((N, 1), _f32), _nd),
    )(x, We, be2, W00, degp)

    a1 = _edge_kernel(g1, src_h, dst_h, zero_nd).reshape(NC, NP, D)

    g2 = pl.pallas_call(
        functools.partial(_tc_mid_body, False), out_shape=_nd,
    )(a1, g1, dinv, b00_2, W01)

    a2 = _edge_kernel(g2, src_h, dst_h, zero_nd).reshape(NC, NP, D)

    g3 = pl.pallas_call(
        functools.partial(_tc_mid_body, True), out_shape=_nd,
    )(a2, g2, dinv, b01_2, W10, bn0g2, bn0b2)

    a3 = _edge_kernel(g3, src_h, dst_h, zero_nd).reshape(NC, NP, D)

    g4 = pl.pallas_call(
        functools.partial(_tc_mid_body, False), out_shape=_nd,
    )(a3, g3, dinv, b10_2, W11)

    a4 = _edge_kernel(g4, src_h, dst_h, zero_nd).reshape(NC, NP, D)

    out = pl.pallas_call(
        _tc_last_body, out_shape=_nd,
    )(a4, g4, dinv, b11_2, bn1g2, bn1b2)
    return out


# speedup vs baseline: 3.5799x; 1.1591x over previous
import functools

import jax
import jax.numpy as jnp
from jax import lax
from jax.experimental import pallas as pl
from jax.experimental.pallas import tpu as pltpu
from jax.experimental.pallas import tpu_sc as plsc

N = 10000
E = 320000
D = 128
NS = 16
NC = 2
NW = NC * NS

NP = N
CHUNK = 125
TILE_E = E // NW
TILE_CHUNKS = TILE_E // CHUNK
HALF_CHUNKS = TILE_CHUNKS // 2

_mesh = plsc.VectorSubcoreMesh(
    core_axis_name="c", subcore_axis_name="s", num_cores=NC, num_subcores=NS
)

_f32 = jnp.float32


@functools.partial(
    pl.kernel,
    out_type=jax.ShapeDtypeStruct((NC, NP), _f32),
    mesh=_mesh,
    scratch_types=[
        pltpu.VMEM((TILE_CHUNKS, CHUNK), jnp.int32),
        pltpu.VMEM((CHUNK,), _f32),
        pltpu.VMEM_SHARED((NP,), _f32),
        pltpu.SemaphoreType.DMA,
    ],
)
def _deg_kernel(dst_hbm, ones_hbm, zero_hbm, out_hbm, idx_v, ones_v, deg_sh, sem):
    cid = lax.axis_index("c")
    sid = lax.axis_index("s")
    wid = cid * NS + sid

    @pl.when(sid == 0)
    def _():
        pltpu.sync_copy(zero_hbm, deg_sh)

    pltpu.sync_copy(ones_hbm, ones_v)
    pltpu.sync_copy(dst_hbm.at[wid], idx_v)
    plsc.subcore_barrier()

    def body(j, carry):
        pltpu.sync_copy(ones_v, deg_sh.at[idx_v.at[j]], add=True)
        return carry

    lax.fori_loop(0, TILE_CHUNKS, body, 0)
    plsc.subcore_barrier()

    @pl.when(sid == 0)
    def _():
        pltpu.sync_copy(deg_sh, out_hbm.at[cid])


@functools.partial(
    pl.kernel,
    out_type=jax.ShapeDtypeStruct((NC * NP, D), _f32),
    mesh=_mesh,
    scratch_types=[
        pltpu.VMEM((HALF_CHUNKS, CHUNK), jnp.int32),
        pltpu.VMEM((HALF_CHUNKS, CHUNK), jnp.int32),
        pltpu.VMEM((CHUNK, D), _f32),
        pltpu.VMEM((CHUNK, D), _f32),
        pltpu.VMEM_SHARED((NP, D), _f32),
        pltpu.SemaphoreType.DMA,
        pltpu.SemaphoreType.DMA,
    ],
)
def _edge_kernel(g_hbm, src_hbm, dst_hbm, zero_hbm, out_hbm,
                 src_v, dst_v, rows0_v, rows1_v, acc_sh, s0, s1):
    cid = lax.axis_index("c")
    sid = lax.axis_index("s")
    wid = cid * NS + sid
    sems = (s0, s1)
    rows = (rows0_v, rows1_v)

    @pl.when(sid < 10)
    def _():
        pltpu.sync_copy(zero_hbm.at[pl.ds(sid * 1000, 1000)],
                        acc_sh.at[pl.ds(sid * 1000, 1000)])

    def _gather(j, b):
        pltpu.async_copy(g_hbm.at[src_v.at[j]], rows[b], sems[b])

    def _gwait(b):
        pltpu.make_async_copy(g_hbm.at[src_v.at[0]], rows[b],
                              sems[b]).wait()

    def _scat(j, b):
        pltpu.sync_copy(rows[b], acc_sh.at[dst_v.at[j]], add=True)

    for p in range(2):
        pltpu.sync_copy(src_hbm.at[wid * 2 + p], src_v)
        pltpu.sync_copy(dst_hbm.at[wid * 2 + p], dst_v)
        if p == 0:
            plsc.subcore_barrier()
        _gather(0, 0)

        def body(g, carry):
            j = 2 * g
            _gather(j + 1, 1)
            _gwait(0)
            _scat(j, 0)

            @pl.when(g < HALF_CHUNKS // 2 - 1)
            def _():
                _gather(j + 2, 0)

            _gwait(1)
            _scat(j + 1, 1)
            return carry

        lax.fori_loop(0, HALF_CHUNKS // 2, body, 0)

    plsc.subcore_barrier()

    @pl.when(sid < 10)
    def _():
        pltpu.sync_copy(acc_sh.at[pl.ds(sid * 1000, 1000)],
                        out_hbm.at[pl.ds(cid * NP + sid * 1000, 1000)])


def _tc1_body(x_ref, we_ref, be_ref, w0_ref, degp_ref, dinv_ref, g_ref):
    deg = degp_ref[0, :N] + degp_ref[1, :N] + 1.0
    dinv = lax.rsqrt(deg)
    dinv_ref[...] = dinv
    h0 = jnp.dot(x_ref[...], we_ref[...], preferred_element_type=_f32)
    h0 = h0 + be_ref[...]
    g_ref[...] = dinv * jnp.dot(h0, w0_ref[...], preferred_element_type=_f32)


def _tc_mid_body(has_bn, ap_ref, g_ref, dinv_ref, b_ref, w_ref, *rest):
    if has_bn:
        bng_ref, bnb_ref, o_ref = rest
    else:
        (o_ref,) = rest
    dinv = dinv_ref[...]
    a = ap_ref[0, :N] + ap_ref[1, :N]
    t = dinv * (a + g_ref[...]) + b_ref[...]
    if has_bn:
        t = t * bng_ref[...] + bnb_ref[...]
    t = jnp.maximum(t, 0.0)
    o_ref[...] = dinv * jnp.dot(t, w_ref[...], preferred_element_type=_f32)


def _tc_last_body(ap_ref, g_ref, dinv_ref, b_ref, bng_ref, bnb_ref, out_ref):
    dinv = dinv_ref[...]
    a = ap_ref[0, :N] + ap_ref[1, :N]
    t = dinv * (a + g_ref[...]) + b_ref[...]
    out_ref[...] = t * bng_ref[...] + bnb_ref[...]


_BN_SCALE = 1.0 / (1.0 + 1e-5) ** 0.5

_nd = jax.ShapeDtypeStruct((N, D), _f32)


def kernel(x, edge_index, batch, edge_attr, We, be, W00, b00, W01, b01,
           W10, b10, W11, b11, bn0_g, bn0_b, bn1_g, bn1_b):
    del batch, edge_attr
    src = edge_index[0]
    dst = edge_index[1]

    src_h = src.reshape(NW * 2, HALF_CHUNKS, CHUNK)
    dst_h = dst.reshape(NW * 2, HALF_CHUNKS, CHUNK)
    dst_f = dst.reshape(NW, TILE_CHUNKS, CHUNK)

    ones_c = jnp.ones((CHUNK,), _f32)
    zero_n = jnp.zeros((NP,), _f32)
    zero_nd = jnp.zeros((NP, D), _f32)

    be2 = be.reshape(1, D)
    b00_2 = b00.reshape(1, D)
    b01_2 = b01.reshape(1, D)
    b10_2 = b10.reshape(1, D)
    b11_2 = b11.reshape(1, D)
    bn0g2 = (bn0_g * _BN_SCALE).reshape(1, D)
    bn0b2 = bn0_b.reshape(1, D)
    bn1g2 = (bn1_g * _BN_SCALE).reshape(1, D)
    bn1b2 = bn1_b.reshape(1, D)

    deg_parts = _deg_kernel(dst_f, ones_c, zero_n)
    degp = deg_parts.reshape(NC, NP, 1)

    dinv, g1 = pl.pallas_call(
        _tc1_body,
        out_shape=(jax.ShapeDtypeStruct((N, 1), _f32), _nd),
    )(x, We, be2, W00, degp)

    a1 = _edge_kernel(g1, src_h, dst_h, zero_nd).reshape(NC, NP, D)

    g2 = pl.pallas_call(
        functools.partial(_tc_mid_body, False), out_shape=_nd,
    )(a1, g1, dinv, b00_2, W01)

    a2 = _edge_kernel(g2, src_h, dst_h, zero_nd).reshape(NC, NP, D)

    g3 = pl.pallas_call(
        functools.partial(_tc_mid_body, True), out_shape=_nd,
    )(a2, g2, dinv, b01_2, W10, bn0g2, bn0b2)

    a3 = _edge_kernel(g3, src_h, dst_h, zero_nd).reshape(NC, NP, D)

    g4 = pl.pallas_call(
        functools.partial(_tc_mid_body, False), out_shape=_nd,
    )(a3, g3, dinv, b10_2, W11)

    a4 = _edge_kernel(g4, src_h, dst_h, zero_nd).reshape(NC, NP, D)

    out = pl.pallas_call(
        _tc_last_body, out_shape=_nd,
    )(a4, g4, dinv, b11_2, bn1g2, bn1b2)
    return out
